# baseline (speedup 1.0000x reference)
import functools

import jax
import jax.numpy as jnp
from jax import lax
from jax.experimental import pallas as pl
from jax.experimental.pallas import tpu as pltpu
from jax.experimental.pallas import tpu_sc as plsc

DIM = 64
K = 10
LANES = 16
CHUNK = 64
NW = 32


def _sc_dots(cpk, csel, ppk, psel, npk, nsel, ctab2, xtab2, b):
    bpw = b // NW
    n_chunks = bpw // CHUNK
    crows_pw = bpw // CHUNK
    nrows_pw = bpw * K // 128
    nrows_pc = CHUNK * K // 128
    mesh = plsc.VectorSubcoreMesh(core_axis_name="c", subcore_axis_name="s")
    nc = 2

    @functools.partial(
        pl.kernel,
        out_type=[
            jax.ShapeDtypeStruct((b,), jnp.float32),
            jax.ShapeDtypeStruct((b,), jnp.float32),
        ],
        mesh=mesh,
        compiler_params=pltpu.CompilerParams(needs_layout_passes=False),
        scratch_types=[
            pltpu.VMEM((crows_pw, CHUNK), jnp.int32),
            pltpu.VMEM((crows_pw, CHUNK), jnp.int32),
            pltpu.VMEM((crows_pw, CHUNK), jnp.int32),
            pltpu.VMEM((crows_pw, CHUNK), jnp.int32),
            pltpu.VMEM((nrows_pw, 128), jnp.int32),
            pltpu.VMEM((nrows_pw, 128), jnp.int32),
            pltpu.VMEM((CHUNK, 128), jnp.float32),
            pltpu.VMEM((CHUNK, 128), jnp.float32),
            pltpu.VMEM((CHUNK * K, 128), jnp.float32),
            pltpu.VMEM((CHUNK,), jnp.float32),
            pltpu.VMEM((CHUNK,), jnp.float32),
            pltpu.SemaphoreType.DMA,
        ],
    )
    def sc_kern(cpk_hbm, csel_hbm, ppk_hbm, psel_hbm, npk_hbm, nsel_hbm,
                ctab_hbm, xtab_hbm, pos_out, neg_out,
                cpk_v, csel_v, ppk_v, psel_v, npk_v, nsel_v,
                crow, prow, nrow, posb, negb, sem):
        wid = lax.axis_index("s") * nc + lax.axis_index("c")
        pltpu.sync_copy(cpk_hbm.at[pl.ds(wid * crows_pw, crows_pw)], cpk_v)
        pltpu.sync_copy(csel_hbm.at[pl.ds(wid * crows_pw, crows_pw)], csel_v)
        pltpu.sync_copy(ppk_hbm.at[pl.ds(wid * crows_pw, crows_pw)], ppk_v)
        pltpu.sync_copy(psel_hbm.at[pl.ds(wid * crows_pw, crows_pw)], psel_v)
        pltpu.sync_copy(npk_hbm.at[pl.ds(wid * nrows_pw, nrows_pw)], npk_v)
        pltpu.sync_copy(nsel_hbm.at[pl.ds(wid * nrows_pw, nrows_pw)], nsel_v)
        iota = lax.iota(jnp.int32, LANES)

        for m in range(n_chunks):
            copies = [
                pltpu.async_copy(ctab_hbm.at[cpk_v.at[m]], crow, sem),
                pltpu.async_copy(xtab_hbm.at[ppk_v.at[m]], prow, sem),
            ]
            for j in range(nrows_pc):
                copies.append(pltpu.async_copy(
                    xtab_hbm.at[npk_v.at[m * nrows_pc + j]],
                    nrow.at[pl.ds(j * 128, 128)], sem))
            for c in copies:
                c.wait()

            def group(g, _):
                ev = g * LANES + iota
                hcv = plsc.load_gather(csel_v, [jnp.full_like(ev, m), ev])
                hpv = plsc.load_gather(psel_v, [jnp.full_like(ev, m), ev])
                nrows = []
                hns = []
                for k in range(K):
                    tl = ev * K + k
                    tg = tl + m * (CHUNK * K)
                    hns.append(plsc.load_gather(
                        nsel_v, [tg >> 7, tg & 127]))
                    nrows.append(tl)

                def dstep(d, acc):
                    pacc, nacc = acc
                    cd = plsc.load_gather(crow, [ev, hcv + d])
                    pd_ = plsc.load_gather(prow, [ev, hpv + d])
                    nsd = plsc.load_gather(nrow, [nrows[0], hns[0] + d])
                    for k in range(1, K):
                        nsd = nsd + plsc.load_gather(
                            nrow, [nrows[k], hns[k] + d])
                    return pacc + cd * pd_, nacc + cd * nsd

                z = jnp.zeros((LANES,), jnp.float32)
                pacc, nacc = lax.fori_loop(0, DIM, dstep, (z, z), unroll=4)
                posb[pl.ds(g * LANES, LANES)] = pacc
                negb[pl.ds(g * LANES, LANES)] = nacc
                return _

            lax.fori_loop(0, CHUNK // LANES, group, 0)
            base = wid * bpw + m * CHUNK
            pltpu.sync_copy(posb, pos_out.at[pl.ds(base, CHUNK)])
            pltpu.sync_copy(negb, neg_out.at[pl.ds(base, CHUNK)])

    return sc_kern(cpk, csel, ppk, psel, npk, nsel, ctab2, xtab2)


def _log_sigmoid(x):
    return jnp.minimum(x, 0.0) - jnp.log1p(jnp.exp(-jnp.abs(x)))


def _tc_reduce_body(pos_ref, neg_ref, out_ref):
    tot = (jnp.sum(_log_sigmoid(pos_ref[...]))
           + jnp.sum(_log_sigmoid(-neg_ref[...])))
    out_ref[0, 0] = tot


def kernel(center, positive_context, negative_context, batch_size,
           center_table, context_table):
    b = center.shape[0]
    cidx = center.astype(jnp.int32)
    pidx = positive_context.astype(jnp.int32)
    nidx = negative_context.astype(jnp.int32).reshape(-1)
    cpk = (cidx >> 1).reshape(b // CHUNK, CHUNK)
    csel = ((cidx & 1) * DIM).reshape(b // CHUNK, CHUNK)
    ppk = (pidx >> 1).reshape(b // CHUNK, CHUNK)
    psel = ((pidx & 1) * DIM).reshape(b // CHUNK, CHUNK)
    npk = (nidx >> 1).reshape(b * K // 128, 128)
    nsel = ((nidx & 1) * DIM).reshape(b * K // 128, 128)
    ctab2 = center_table.reshape(-1, 128)
    xtab2 = context_table.reshape(-1, 128)

    pos_dot, neg_dot = _sc_dots(
        cpk, csel, ppk, psel, npk, nsel, ctab2, xtab2, b)

    tot = pl.pallas_call(
        _tc_reduce_body,
        out_shape=jax.ShapeDtypeStruct((1, 1), jnp.float32),
        out_specs=pl.BlockSpec(memory_space=pltpu.SMEM),
    )(pos_dot.reshape(b // 128, 128), neg_dot.reshape(b // 128, 128))
    return -tot[0, 0] / batch_size


# speedup vs baseline: 1.0009x; 1.0009x over previous
import functools

import jax
import jax.numpy as jnp
from jax import lax
from jax.experimental import pallas as pl
from jax.experimental.pallas import tpu as pltpu
from jax.experimental.pallas import tpu_sc as plsc

DIM = 64
K = 10
LANES = 16
CHUNK = 64
NW = 32


def _sc_dots(cpk, csel, ppk, psel, npk, nsel, ctab2, xtab2, b):
    bpw = b // NW
    n_chunks = bpw // CHUNK
    crows_pw = bpw // CHUNK
    nrows_pw = bpw * K // 128
    nrows_pc = CHUNK * K // 128
    mesh = plsc.VectorSubcoreMesh(core_axis_name="c", subcore_axis_name="s")
    nc = 2

    @functools.partial(
        pl.kernel,
        out_type=[
            jax.ShapeDtypeStruct((b,), jnp.float32),
            jax.ShapeDtypeStruct((b,), jnp.float32),
        ],
        mesh=mesh,
        compiler_params=pltpu.CompilerParams(needs_layout_passes=False),
        scratch_types=[
            pltpu.VMEM((crows_pw, CHUNK), jnp.int32),
            pltpu.VMEM((crows_pw, CHUNK), jnp.int32),
            pltpu.VMEM((crows_pw, CHUNK), jnp.int32),
            pltpu.VMEM((crows_pw, CHUNK), jnp.int32),
            pltpu.VMEM((nrows_pw, 128), jnp.int32),
            pltpu.VMEM((nrows_pw, 128), jnp.int32),
            pltpu.VMEM((CHUNK, 128), jnp.float32),
            pltpu.VMEM((CHUNK, 128), jnp.float32),
            pltpu.VMEM((CHUNK * K, 128), jnp.float32),
            pltpu.VMEM((CHUNK,), jnp.float32),
            pltpu.VMEM((CHUNK,), jnp.float32),
            pltpu.SemaphoreType.DMA,
        ],
    )
    def sc_kern(cpk_hbm, csel_hbm, ppk_hbm, psel_hbm, npk_hbm, nsel_hbm,
                ctab_hbm, xtab_hbm, pos_out, neg_out,
                cpk_v, csel_v, ppk_v, psel_v, npk_v, nsel_v,
                crow, prow, nrow, posb, negb, sem):
        wid = lax.axis_index("s") * nc + lax.axis_index("c")
        pltpu.sync_copy(cpk_hbm.at[pl.ds(wid * crows_pw, crows_pw)], cpk_v)
        pltpu.sync_copy(csel_hbm.at[pl.ds(wid * crows_pw, crows_pw)], csel_v)
        pltpu.sync_copy(ppk_hbm.at[pl.ds(wid * crows_pw, crows_pw)], ppk_v)
        pltpu.sync_copy(psel_hbm.at[pl.ds(wid * crows_pw, crows_pw)], psel_v)
        pltpu.sync_copy(npk_hbm.at[pl.ds(wid * nrows_pw, nrows_pw)], npk_v)
        pltpu.sync_copy(nsel_hbm.at[pl.ds(wid * nrows_pw, nrows_pw)], nsel_v)
        iota = lax.iota(jnp.int32, LANES)

        for m in range(n_chunks):
            copies = [
                pltpu.async_copy(ctab_hbm.at[cpk_v.at[m]], crow, sem),
                pltpu.async_copy(xtab_hbm.at[ppk_v.at[m]], prow, sem),
            ]
            for j in range(nrows_pc):
                copies.append(pltpu.async_copy(
                    xtab_hbm.at[npk_v.at[m * nrows_pc + j]],
                    nrow.at[pl.ds(j * 128, 128)], sem))
            for c in copies:
                c.wait()

            def group(g, _):
                ev = g * LANES + iota
                hcv = plsc.load_gather(csel_v, [jnp.full_like(ev, m), ev])
                hpv = plsc.load_gather(psel_v, [jnp.full_like(ev, m), ev])
                nrows = []
                hns = []
                for k in range(K):
                    tl = ev * K + k
                    tg = tl + m * (CHUNK * K)
                    hns.append(plsc.load_gather(
                        nsel_v, [tg >> 7, tg & 127]))
                    nrows.append(tl)

                def dstep(d, acc):
                    pacc, nacc = acc
                    cd = plsc.load_gather(crow, [ev, hcv + d])
                    pd_ = plsc.load_gather(prow, [ev, hpv + d])
                    nsd = plsc.load_gather(nrow, [nrows[0], hns[0] + d])
                    for k in range(1, K):
                        nsd = nsd + plsc.load_gather(
                            nrow, [nrows[k], hns[k] + d])
                    return pacc + cd * pd_, nacc + cd * nsd

                z = jnp.zeros((LANES,), jnp.float32)
                pacc, nacc = lax.fori_loop(0, DIM, dstep, (z, z), unroll=4)
                posb[pl.ds(g * LANES, LANES)] = pacc
                negb[pl.ds(g * LANES, LANES)] = nacc
                return _

            lax.fori_loop(0, CHUNK // LANES, group, 0)
            base = wid * bpw + m * CHUNK
            pltpu.sync_copy(posb, pos_out.at[pl.ds(base, CHUNK)])
            pltpu.sync_copy(negb, neg_out.at[pl.ds(base, CHUNK)])

    return sc_kern(cpk, csel, ppk, psel, npk, nsel, ctab2, xtab2)


def _log_sigmoid(x):
    return jnp.minimum(x, 0.0) - jnp.log1p(jnp.exp(-jnp.abs(x)))


def _tc_reduce_body(pos_ref, neg_ref, out_ref):
    tot = (jnp.sum(_log_sigmoid(pos_ref[...]))
           + jnp.sum(_log_sigmoid(-neg_ref[...])))
    out_ref[0, 0] = tot


def kernel(center, positive_context, negative_context, batch_size,
           center_table, context_table):
    b = center.shape[0]
    v = center_table.shape[0]
    cidx = center.astype(jnp.int32)
    pidx = positive_context.astype(jnp.int32)
    nidx = negative_context.astype(jnp.int32).reshape(-1)
    cpk = (cidx >> 1).reshape(b // CHUNK, CHUNK)
    csel = ((cidx & 1) * DIM).reshape(b // CHUNK, CHUNK)
    ppk = (pidx >> 1).reshape(b // CHUNK, CHUNK)
    psel = ((pidx & 1) * DIM).reshape(b // CHUNK, CHUNK)
    npk = (nidx >> 1).reshape(b * K // 128, 128)
    nsel = ((nidx & 1) * DIM).reshape(b * K // 128, 128)
    ctab2 = center_table.reshape(v // 2, 2 * DIM)
    xtab2 = context_table.reshape(v // 2, 2 * DIM)

    pos_dot, neg_dot = _sc_dots(
        cpk, csel, ppk, psel, npk, nsel, ctab2, xtab2, b)

    tot = pl.pallas_call(
        _tc_reduce_body,
        out_shape=jax.ShapeDtypeStruct((1, 1), jnp.float32),
        out_specs=pl.BlockSpec(memory_space=pltpu.SMEM),
    )(pos_dot.reshape(b // 128, 128), neg_dot.reshape(b // 128, 128))
    return -tot[0, 0] / batch_size


# speedup vs baseline: 1.0883x; 1.0873x over previous
import functools

import jax
import jax.numpy as jnp
from jax import lax
from jax.experimental import pallas as pl
from jax.experimental.pallas import tpu as pltpu
from jax.experimental.pallas import tpu_sc as plsc

DIM = 64
K = 10
LANES = 16
CHUNK = 64
NW = 32


def _sc_dots(cidx, pidx, nidx, ctab, xtab, b):
    bpw = b // NW
    n_chunks = bpw // CHUNK
    crows_pw = bpw // CHUNK
    nrows_pw = bpw * K // 128
    nrows_pc = CHUNK * K // 128
    mesh = plsc.VectorSubcoreMesh(core_axis_name="c", subcore_axis_name="s")
    nc = 2

    @functools.partial(
        pl.kernel,
        out_type=[
            jax.ShapeDtypeStruct((b,), jnp.float32),
            jax.ShapeDtypeStruct((b,), jnp.float32),
        ],
        mesh=mesh,
        compiler_params=pltpu.CompilerParams(needs_layout_passes=False),
        scratch_types=[
            pltpu.VMEM((crows_pw, CHUNK), jnp.int32),
            pltpu.VMEM((crows_pw, CHUNK), jnp.int32),
            pltpu.VMEM((nrows_pw, 128), jnp.int32),
            pltpu.VMEM((CHUNK, 128), jnp.float32),
            pltpu.VMEM((CHUNK, 128), jnp.float32),
            pltpu.VMEM((CHUNK * K, 128), jnp.float32),
            pltpu.VMEM((CHUNK,), jnp.float32),
            pltpu.VMEM((CHUNK,), jnp.float32),
            pltpu.SemaphoreType.DMA,
        ],
    )
    def sc_kern(cidx_hbm, pidx_hbm, nidx_hbm, ctab_hbm, xtab_hbm,
                pos_out, neg_out,
                cidx_v, pidx_v, nidx_v, crow, prow, nrow, posb, negb, sem):
        wid = lax.axis_index("s") * nc + lax.axis_index("c")
        pltpu.sync_copy(cidx_hbm.at[pl.ds(wid * crows_pw, crows_pw)], cidx_v)
        pltpu.sync_copy(pidx_hbm.at[pl.ds(wid * crows_pw, crows_pw)], pidx_v)
        pltpu.sync_copy(nidx_hbm.at[pl.ds(wid * nrows_pw, nrows_pw)], nidx_v)
        iota = lax.iota(jnp.int32, LANES)

        for m in range(n_chunks):
            copies = [
                pltpu.async_copy(ctab_hbm.at[cidx_v.at[m]], crow, sem),
                pltpu.async_copy(xtab_hbm.at[pidx_v.at[m]], prow, sem),
            ]
            for j in range(nrows_pc):
                copies.append(pltpu.async_copy(
                    xtab_hbm.at[nidx_v.at[m * nrows_pc + j]],
                    nrow.at[pl.ds(j * 128, 128)], sem))
            for c in copies:
                c.wait()

            def group(g, _):
                ev = g * LANES + iota
                nrows = [ev * K + k for k in range(K)]

                def dstep(d, acc):
                    pacc, nacc = acc
                    dv = jnp.zeros((LANES,), jnp.int32) + d
                    cd = plsc.load_gather(crow, [ev, dv])
                    pd_ = plsc.load_gather(prow, [ev, dv])
                    nsd = plsc.load_gather(nrow, [nrows[0], dv])
                    for k in range(1, K):
                        nsd = nsd + plsc.load_gather(nrow, [nrows[k], dv])
                    return pacc + cd * pd_, nacc + cd * nsd

                z = jnp.zeros((LANES,), jnp.float32)
                pacc, nacc = lax.fori_loop(0, DIM, dstep, (z, z), unroll=4)
                posb[pl.ds(g * LANES, LANES)] = pacc
                negb[pl.ds(g * LANES, LANES)] = nacc
                return _

            lax.fori_loop(0, CHUNK // LANES, group, 0)
            base = wid * bpw + m * CHUNK
            pltpu.sync_copy(posb, pos_out.at[pl.ds(base, CHUNK)])
            pltpu.sync_copy(negb, neg_out.at[pl.ds(base, CHUNK)])

    return sc_kern(cidx, pidx, nidx, ctab, xtab)


def _log_sigmoid(x):
    return jnp.minimum(x, 0.0) - jnp.log1p(jnp.exp(-jnp.abs(x)))


def _tc_reduce_body(pos_ref, neg_ref, out_ref):
    tot = (jnp.sum(_log_sigmoid(pos_ref[...]))
           + jnp.sum(_log_sigmoid(-neg_ref[...])))
    out_ref[0, 0] = tot


def kernel(center, positive_context, negative_context, batch_size,
           center_table, context_table):
    b = center.shape[0]
    cidx = center.astype(jnp.int32).reshape(b // CHUNK, CHUNK)
    pidx = positive_context.astype(jnp.int32).reshape(b // CHUNK, CHUNK)
    nidx = negative_context.astype(jnp.int32).reshape(b * K // 128, 128)
    ctab = jnp.pad(center_table, ((0, 0), (0, 128 - DIM)))
    xtab = jnp.pad(context_table, ((0, 0), (0, 128 - DIM)))

    pos_dot, neg_dot = _sc_dots(cidx, pidx, nidx, ctab, xtab, b)

    tot = pl.pallas_call(
        _tc_reduce_body,
        out_shape=jax.ShapeDtypeStruct((1, 1), jnp.float32),
        out_specs=pl.BlockSpec(memory_space=pltpu.SMEM),
    )(pos_dot.reshape(b // 128, 128), neg_dot.reshape(b // 128, 128))
    return -tot[0, 0] / batch_size
